# baseline (speedup 1.0000x reference)
import functools

import jax
import jax.numpy as jnp
from jax import lax
from jax.experimental import pallas as pl
from jax.experimental.pallas import tpu as pltpu
from jax.experimental.pallas import tpu_sc as plsc

_NROWS = 16384
_NCOLS = 16


@functools.cache
def _build():
    info = plsc.get_sparse_core_info()
    nc, ns, lanes = info.num_cores, info.num_subcores, info.num_lanes
    nw = nc * ns
    rows_per_w = _NROWS // nw
    nblk = rows_per_w // lanes
    half_rows = rows_per_w // 2
    half_elems = half_rows * _NCOLS

    mesh = plsc.VectorSubcoreMesh(core_axis_name="c", subcore_axis_name="s")

    @functools.partial(
        pl.kernel,
        mesh=mesh,
        out_type=jax.ShapeDtypeStruct((_NROWS,), jnp.float32),
        compiler_params=pltpu.CompilerParams(needs_layout_passes=False),
        scratch_types=[
            pltpu.VMEM((rows_per_w * _NCOLS,), jnp.float32),
            pltpu.VMEM((lanes,), jnp.int32),
            pltpu.VMEM((lanes,), jnp.float32),
            pltpu.VMEM((rows_per_w,), jnp.float32),
            pltpu.VMEM((lanes,), jnp.float32),
            pltpu.SemaphoreType.DMA,
            pltpu.SemaphoreType.DMA,
            pltpu.SemaphoreType.DMA,
        ],
    )
    def sc_kernel(
        x_hbm, keys_hbm, vals_hbm, out_hbm, x_v, keys_v, vals_v, out_v, pay_v,
        sem0, sem1, sem2,
    ):
        wid = lax.axis_index("s") * nc + lax.axis_index("c")
        base = wid * rows_per_w
        e0 = base * _NCOLS
        cp0 = pltpu.async_copy(
            x_hbm.at[pl.ds(e0, half_elems)], x_v.at[pl.ds(0, half_elems)], sem0
        )
        cp1 = pltpu.async_copy(
            x_hbm.at[pl.ds(e0 + half_elems, half_elems)],
            x_v.at[pl.ds(half_elems, half_elems)],
            sem1,
        )
        cpk = pltpu.async_copy(keys_hbm, keys_v, sem2)
        cpv = pltpu.async_copy(vals_hbm, vals_v, sem2)

        lane = lax.iota(jnp.int32, lanes)
        lane16 = lane * _NCOLS

        cpk.wait()
        cpv.wait()
        keys_vec = keys_v[pl.ds(0, lanes)]
        vals_vec = vals_v[pl.ds(0, lanes)]
        pay_v[pl.ds(0, lanes)] = jnp.where(
            keys_vec == lane, vals_vec, jnp.full((lanes,), -1.0, jnp.float32)
        )

        def block(blk):
            base_i = lane16 + blk * (lanes * _NCOLS)
            cols = [(lane + j) & (_NCOLS - 1) for j in range(_NCOLS)]
            diags = [
                plsc.load_gather(x_v, [base_i + cols[j]]) for j in range(_NCOLS)
            ]
            mx = list(diags)
            while len(mx) > 1:
                mx = [
                    jnp.maximum(mx[a], mx[a + 1]) for a in range(0, len(mx), 2)
                ]
            vmax = mx[0]
            sentinel = jnp.full((lanes,), _NCOLS, jnp.int32)
            cand = [
                jnp.where(diags[j] == vmax, cols[j], sentinel)
                for j in range(_NCOLS)
            ]
            while len(cand) > 1:
                cand = [
                    jnp.minimum(cand[a], cand[a + 1])
                    for a in range(0, len(cand), 2)
                ]
            out_v[pl.ds(blk * lanes, lanes)] = plsc.load_gather(pay_v, [cand[0]])

        cp0.wait()
        for blk in range(nblk // 2):
            block(blk)
        cp1.wait()
        for blk in range(nblk // 2, nblk):
            block(blk)

        pltpu.sync_copy(out_v, out_hbm.at[pl.ds(base, rows_per_w)])

    return sc_kernel


def kernel(tensor_input, keys, values):
    keys32 = keys.astype(jnp.int32)
    vals32 = values.astype(jnp.float32)
    x_flat = tensor_input.reshape(-1)
    return _build()(x_flat, keys32, vals32)


# speedup vs baseline: 1.5723x; 1.5723x over previous
import functools

import jax
import jax.numpy as jnp
from jax import lax
from jax.experimental import pallas as pl
from jax.experimental.pallas import tpu as pltpu
from jax.experimental.pallas import tpu_sc as plsc

_NROWS = 16384
_NCOLS = 16


@functools.cache
def _build():
    info = plsc.get_sparse_core_info()
    nc, ns, lanes = info.num_cores, info.num_subcores, info.num_lanes
    nw = nc * ns
    rows_per_w = _NROWS // nw
    nblk = rows_per_w // lanes
    half_rows = rows_per_w // 2
    half_elems = half_rows * _NCOLS

    mesh = plsc.VectorSubcoreMesh(core_axis_name="c", subcore_axis_name="s")

    @functools.partial(
        pl.kernel,
        mesh=mesh,
        out_type=jax.ShapeDtypeStruct((_NROWS,), jnp.float32),
        compiler_params=pltpu.CompilerParams(needs_layout_passes=False),
        scratch_types=[
            pltpu.VMEM((rows_per_w * _NCOLS,), jnp.float32),
            pltpu.VMEM((lanes,), jnp.int32),
            pltpu.VMEM((lanes,), jnp.float32),
            pltpu.VMEM((rows_per_w,), jnp.float32),
            pltpu.VMEM((lanes,), jnp.float32),
            pltpu.SemaphoreType.DMA,
            pltpu.SemaphoreType.DMA,
            pltpu.SemaphoreType.DMA,
        ],
    )
    def sc_kernel(
        x_hbm, keys_hbm, vals_hbm, out_hbm, x_v, keys_v, vals_v, out_v, pay_v,
        sem0, sem1, sem2,
    ):
        wid = lax.axis_index("s") * nc + lax.axis_index("c")
        base = wid * rows_per_w
        e0 = base * _NCOLS
        cp0 = pltpu.async_copy(
            x_hbm.at[pl.ds(e0, half_elems)], x_v.at[pl.ds(0, half_elems)], sem0
        )
        cp1 = pltpu.async_copy(
            x_hbm.at[pl.ds(e0 + half_elems, half_elems)],
            x_v.at[pl.ds(half_elems, half_elems)],
            sem1,
        )
        cpk = pltpu.async_copy(keys_hbm, keys_v, sem2)
        cpv = pltpu.async_copy(vals_hbm, vals_v, sem2)

        lane = lax.iota(jnp.int32, lanes)
        lane16 = lane * _NCOLS

        cpk.wait()
        cpv.wait()
        keys_vec = keys_v[pl.ds(0, lanes)]
        vals_vec = vals_v[pl.ds(0, lanes)]
        pay_v[pl.ds(0, lanes)] = jnp.where(
            keys_vec == lane, vals_vec, jnp.full((lanes,), -1.0, jnp.float32)
        )

        def block(blk):
            base_i = lane16 + blk * (lanes * _NCOLS)
            cols = [(lane + j) & (_NCOLS - 1) for j in range(_NCOLS)]
            diags = [
                plsc.load_gather(x_v, [base_i + cols[j]]) for j in range(_NCOLS)
            ]
            mx = list(diags)
            while len(mx) > 1:
                mx = [
                    jnp.maximum(mx[a], mx[a + 1]) for a in range(0, len(mx), 2)
                ]
            vmax = mx[0]
            sentinel = jnp.full((lanes,), _NCOLS, jnp.int32)
            cand = [
                jnp.where(diags[j] == vmax, cols[j], sentinel)
                for j in range(_NCOLS)
            ]
            while len(cand) > 1:
                cand = [
                    jnp.minimum(cand[a], cand[a + 1])
                    for a in range(0, len(cand), 2)
                ]
            out_v[pl.ds(blk * lanes, lanes)] = plsc.load_gather(pay_v, [cand[0]])

        cp0.wait()

        @plsc.parallel_loop(0, nblk // 2, unroll=4)
        def _loop_a(blk):
            block(blk)

        cpo0 = pltpu.async_copy(
            out_v.at[pl.ds(0, half_rows)], out_hbm.at[pl.ds(base, half_rows)], sem2
        )
        cp1.wait()

        @plsc.parallel_loop(nblk // 2, nblk, unroll=4)
        def _loop_b(blk):
            block(blk)

        cpo0.wait()
        pltpu.sync_copy(
            out_v.at[pl.ds(half_rows, half_rows)],
            out_hbm.at[pl.ds(base + half_rows, half_rows)],
        )

    return sc_kernel


def kernel(tensor_input, keys, values):
    keys32 = keys.astype(jnp.int32)
    vals32 = values.astype(jnp.float32)
    x_flat = tensor_input.reshape(-1)
    return _build()(x_flat, keys32, vals32)


# speedup vs baseline: 1.5754x; 1.0020x over previous
import functools

import jax
import jax.numpy as jnp
from jax import lax
from jax.experimental import pallas as pl
from jax.experimental.pallas import tpu as pltpu
from jax.experimental.pallas import tpu_sc as plsc

_NROWS = 16384
_NCOLS = 16


@functools.cache
def _build():
    info = plsc.get_sparse_core_info()
    nc, ns, lanes = info.num_cores, info.num_subcores, info.num_lanes
    nw = nc * ns
    rows_per_w = _NROWS // nw
    nblk = rows_per_w // lanes
    half_rows = rows_per_w // 2
    half_elems = half_rows * _NCOLS

    mesh = plsc.VectorSubcoreMesh(core_axis_name="c", subcore_axis_name="s")

    @functools.partial(
        pl.kernel,
        mesh=mesh,
        out_type=jax.ShapeDtypeStruct((_NROWS,), jnp.float32),
        compiler_params=pltpu.CompilerParams(needs_layout_passes=False),
        scratch_types=[
            pltpu.VMEM((rows_per_w * _NCOLS,), jnp.float32),
            pltpu.VMEM((lanes,), jnp.int32),
            pltpu.VMEM((lanes,), jnp.float32),
            pltpu.VMEM((rows_per_w,), jnp.float32),
            pltpu.VMEM((lanes,), jnp.float32),
            pltpu.SemaphoreType.DMA,
            pltpu.SemaphoreType.DMA,
            pltpu.SemaphoreType.DMA,
        ],
    )
    def sc_kernel(
        x_hbm, keys_hbm, vals_hbm, out_hbm, x_v, keys_v, vals_v, out_v, pay_v,
        sem0, sem1, sem2,
    ):
        wid = lax.axis_index("s") * nc + lax.axis_index("c")
        base = wid * rows_per_w
        e0 = base * _NCOLS
        cp0 = pltpu.async_copy(
            x_hbm.at[pl.ds(e0, half_elems)], x_v.at[pl.ds(0, half_elems)], sem0
        )
        cp1 = pltpu.async_copy(
            x_hbm.at[pl.ds(e0 + half_elems, half_elems)],
            x_v.at[pl.ds(half_elems, half_elems)],
            sem1,
        )
        cpk = pltpu.async_copy(keys_hbm, keys_v, sem2)
        cpv = pltpu.async_copy(vals_hbm, vals_v, sem2)

        lane = lax.iota(jnp.int32, lanes)
        lane16 = lane * _NCOLS

        cpk.wait()
        cpv.wait()
        keys_vec = keys_v[pl.ds(0, lanes)]
        vals_vec = vals_v[pl.ds(0, lanes)]
        pay_v[pl.ds(0, lanes)] = jnp.where(
            keys_vec == lane, vals_vec, jnp.full((lanes,), -1.0, jnp.float32)
        )

        def block(blk):
            base_i = lane16 + blk * (lanes * _NCOLS)
            cols = [(lane + j) & (_NCOLS - 1) for j in range(_NCOLS)]
            diags = [
                plsc.load_gather(x_v, [base_i + cols[j]]) for j in range(_NCOLS)
            ]
            mx = list(diags)
            while len(mx) > 1:
                mx = [
                    jnp.maximum(mx[a], mx[a + 1]) for a in range(0, len(mx), 2)
                ]
            vmax = mx[0]
            sentinel = jnp.full((lanes,), _NCOLS - 1, jnp.int32)
            cand = [
                jnp.where(diags[j] == vmax, cols[j], sentinel)
                for j in range(_NCOLS)
            ]
            while len(cand) > 1:
                cand = [
                    jnp.minimum(cand[a], cand[a + 1])
                    for a in range(0, len(cand), 2)
                ]
            out_v[pl.ds(blk * lanes, lanes)] = plsc.load_gather(pay_v, [cand[0]])

        cp0.wait()

        @plsc.parallel_loop(0, nblk // 2, unroll=4)
        def _loop_a(blk):
            block(blk)

        cpo0 = pltpu.async_copy(
            out_v.at[pl.ds(0, half_rows)], out_hbm.at[pl.ds(base, half_rows)], sem2
        )
        cp1.wait()

        @plsc.parallel_loop(nblk // 2, nblk, unroll=4)
        def _loop_b(blk):
            block(blk)

        cpo0.wait()
        pltpu.sync_copy(
            out_v.at[pl.ds(half_rows, half_rows)],
            out_hbm.at[pl.ds(base + half_rows, half_rows)],
        )

    return sc_kernel


def kernel(tensor_input, keys, values):
    keys32 = keys.astype(jnp.int32)
    vals32 = values.astype(jnp.float32)
    x_flat = tensor_input.reshape(-1)
    return _build()(x_flat, keys32, vals32)


# speedup vs baseline: 1.5952x; 1.0126x over previous
import functools

import jax
import jax.numpy as jnp
from jax import lax
from jax.experimental import pallas as pl
from jax.experimental.pallas import tpu as pltpu
from jax.experimental.pallas import tpu_sc as plsc

_NROWS = 16384
_NCOLS = 16


@functools.cache
def _build():
    info = plsc.get_sparse_core_info()
    nc, ns, lanes = info.num_cores, info.num_subcores, info.num_lanes
    nw = nc * ns
    rows_per_w = _NROWS // nw
    nblk = rows_per_w // lanes
    half_rows = rows_per_w // 2
    half_elems = half_rows * _NCOLS

    mesh = plsc.VectorSubcoreMesh(core_axis_name="c", subcore_axis_name="s")

    @functools.partial(
        pl.kernel,
        mesh=mesh,
        out_type=jax.ShapeDtypeStruct((_NROWS,), jnp.float32),
        compiler_params=pltpu.CompilerParams(needs_layout_passes=False),
        scratch_types=[
            pltpu.VMEM((rows_per_w * _NCOLS,), jnp.float32),
            pltpu.VMEM((lanes,), jnp.int32),
            pltpu.VMEM((lanes,), jnp.float32),
            pltpu.VMEM((rows_per_w,), jnp.float32),
            pltpu.VMEM((lanes,), jnp.float32),
            pltpu.SemaphoreType.DMA,
            pltpu.SemaphoreType.DMA,
            pltpu.SemaphoreType.DMA,
        ],
    )
    def sc_kernel(
        x_hbm, keys_hbm, vals_hbm, out_hbm, x_v, keys_v, vals_v, out_v, pay_v,
        sem0, sem1, sem2,
    ):
        wid = lax.axis_index("s") * nc + lax.axis_index("c")
        base = wid * rows_per_w
        e0 = base * _NCOLS
        cp0 = pltpu.async_copy(
            x_hbm.at[pl.ds(e0, half_elems)], x_v.at[pl.ds(0, half_elems)], sem0
        )
        cp1 = pltpu.async_copy(
            x_hbm.at[pl.ds(e0 + half_elems, half_elems)],
            x_v.at[pl.ds(half_elems, half_elems)],
            sem1,
        )
        cpk = pltpu.async_copy(keys_hbm, keys_v, sem2)
        cpv = pltpu.async_copy(vals_hbm, vals_v, sem2)

        lane = lax.iota(jnp.int32, lanes)
        lane16 = lane * _NCOLS

        cpk.wait()
        cpv.wait()
        keys_vec = keys_v[pl.ds(0, lanes)]
        vals_vec = vals_v[pl.ds(0, lanes)]
        pay_v[pl.ds(0, lanes)] = jnp.where(
            keys_vec == lane, vals_vec, jnp.full((lanes,), -1.0, jnp.float32)
        )

        def block(blk):
            base_i = lane16 + blk * (lanes * _NCOLS)
            cols = [(lane + j) & (_NCOLS - 1) for j in range(_NCOLS)]
            diags = [
                plsc.load_gather(x_v, [base_i + cols[j]]) for j in range(_NCOLS)
            ]
            mx = list(diags)
            while len(mx) > 1:
                mx = [
                    jnp.maximum(mx[a], mx[a + 1]) for a in range(0, len(mx), 2)
                ]
            vmax = mx[0]
            sentinel = jnp.full((lanes,), _NCOLS - 1, jnp.int32)
            cand = [
                jnp.where(diags[j] == vmax, cols[j], sentinel)
                for j in range(_NCOLS)
            ]
            while len(cand) > 1:
                cand = [
                    jnp.minimum(cand[a], cand[a + 1])
                    for a in range(0, len(cand), 2)
                ]
            out_v[pl.ds(blk * lanes, lanes)] = plsc.load_gather(pay_v, [cand[0]])

        def body0(blk, carry):
            block(blk)
            return carry

        cp0.wait()
        lax.fori_loop(0, nblk // 2, body0, 0)
        cp1.wait()
        lax.fori_loop(nblk // 2, nblk, body0, 0)
        pltpu.sync_copy(out_v, out_hbm.at[pl.ds(base, rows_per_w)])

    return sc_kernel


def kernel(tensor_input, keys, values):
    keys32 = keys.astype(jnp.int32)
    vals32 = values.astype(jnp.float32)
    x_flat = tensor_input.reshape(-1)
    return _build()(x_flat, keys32, vals32)
